# baseline (speedup 1.0000x reference)
import functools

import jax
import jax.numpy as jnp
from jax import lax
from jax.experimental import pallas as pl
from jax.experimental.pallas import tpu as pltpu
from jax.experimental.pallas import tpu_sc as plsc

N = 512
D = 64
NCORE = 2
DSPLIT = D // NCORE
NSUB = 16
CHUNK = N // NSUB
LANES = 16
TILE = 128
BATCH = 8


def _sc_body(row_hbm, refs_hbm, embt_hbm, out_hbm,
             refs_v, rowbuf_v, ids_v, rows_v, tmp_v, out32_v,
             blk0, blk1, blk2, blk3, blk4, blk5, blk6, blk7,
             blk8, blk9, blk10, blk11, blk12, blk13, blk14, blk15,
             acc_sh, sem, sem_b):
    blksets = [[blk0, blk1, blk2, blk3, blk4, blk5, blk6, blk7],
               [blk8, blk9, blk10, blk11, blk12, blk13, blk14, blk15]]
    sems = [sem, sem_b]
    cid = lax.axis_index("c")
    sid = lax.axis_index("s")
    base = sid * CHUNK
    dlo = pl.multiple_of(cid * DSPLIT, DSPLIT)

    pltpu.sync_copy(refs_hbm.at[pl.ds(base, CHUNK)], refs_v)

    pltpu.async_copy(row_hbm.at[refs_v], rowbuf_v, sem).wait()
    for c in range(CHUNK // LANES):
        lane = lax.iota(jnp.int32, LANES)
        row_idx = lane + jnp.int32(c * LANES)
        col_idx = row_idx + base
        ids_v[pl.ds(LANES + c * LANES, LANES)] = plsc.load_gather(
            rowbuf_v, [row_idx, col_idx])

    ngroups = CHUNK // BATCH

    def fire(g):
        voffs, copies = [], []
        for b in range(BATCH):
            r = g * BATCH + b
            vid = ids_v[pl.ds(LANES + r, LANES)][0]
            vblk = vid // TILE
            voffs.append(vid - vblk * TILE)
            start = pl.multiple_of(vblk * TILE, TILE)
            copies.append(pltpu.async_copy(
                embt_hbm.at[pl.ds(dlo, DSPLIT), pl.ds(start, TILE)],
                blksets[g % 2][b], sems[g % 2]))
        return voffs, copies

    inflight = fire(0)
    for g in range(ngroups):
        voffs, copies = inflight
        if g + 1 < ngroups:
            nxt = fire(g + 1)
        for cp in copies:
            cp.wait()
        for b in range(BATCH):
            r = g * BATCH + b
            voff_vec = jnp.full((LANES,), voffs[b], jnp.int32)
            for c in range(DSPLIT // LANES):
                dvec = lax.iota(jnp.int32, LANES) + jnp.int32(c * LANES)
                rows_v[r, pl.ds(c * LANES, LANES)] = plsc.load_gather(
                    blksets[g % 2][b], [dvec, voff_vec])
            for c in range(DSPLIT // LANES, TILE // LANES):
                rows_v[r, pl.ds(c * LANES, LANES)] = jnp.zeros(
                    (LANES,), jnp.float32)
        if g + 1 < ngroups:
            inflight = nxt

    for r in range(CHUNK):
        for c in range(TILE // LANES):
            tmp_v[r, pl.ds(c * LANES, LANES)] = jnp.zeros((LANES,), jnp.float32)
    pltpu.sync_copy(tmp_v, acc_sh.at[pl.ds(base, CHUNK)])
    plsc.subcore_barrier()

    pltpu.sync_copy(rows_v, acc_sh.at[refs_v], add=True)
    plsc.subcore_barrier()

    pltpu.sync_copy(acc_sh.at[pl.ds(base, CHUNK)], tmp_v)
    for r in range(CHUNK):
        for c in range(DSPLIT // LANES):
            out32_v[r, pl.ds(c * LANES, LANES)] = \
                tmp_v[r, pl.ds(c * LANES, LANES)]
    pltpu.sync_copy(out32_v, out_hbm.at[cid, pl.ds(base, CHUNK)])


def kernel(row, row_refs, row_embeddings):
    mesh = plsc.VectorSubcoreMesh(core_axis_name="c", subcore_axis_name="s")
    k = functools.partial(
        pl.kernel,
        out_type=jax.ShapeDtypeStruct((NCORE, N, DSPLIT), jnp.float32),
        mesh=mesh,
        compiler_params=pltpu.CompilerParams(
            use_tc_tiling_on_sc=True, needs_layout_passes=False,
            disable_bounds_checks=True),
        scratch_types=[
            pltpu.VMEM((CHUNK,), jnp.int32),
            pltpu.VMEM((CHUNK, N), jnp.int32),
            pltpu.VMEM((2 * LANES + CHUNK,), jnp.int32),
            pltpu.VMEM((CHUNK, TILE), jnp.float32),
            pltpu.VMEM((CHUNK, TILE), jnp.float32),
            pltpu.VMEM((CHUNK, DSPLIT), jnp.float32),
            pltpu.VMEM((DSPLIT, TILE), jnp.float32),
            pltpu.VMEM((DSPLIT, TILE), jnp.float32),
            pltpu.VMEM((DSPLIT, TILE), jnp.float32),
            pltpu.VMEM((DSPLIT, TILE), jnp.float32),
            pltpu.VMEM((DSPLIT, TILE), jnp.float32),
            pltpu.VMEM((DSPLIT, TILE), jnp.float32),
            pltpu.VMEM((DSPLIT, TILE), jnp.float32),
            pltpu.VMEM((DSPLIT, TILE), jnp.float32),
            pltpu.VMEM((DSPLIT, TILE), jnp.float32),
            pltpu.VMEM((DSPLIT, TILE), jnp.float32),
            pltpu.VMEM((DSPLIT, TILE), jnp.float32),
            pltpu.VMEM((DSPLIT, TILE), jnp.float32),
            pltpu.VMEM((DSPLIT, TILE), jnp.float32),
            pltpu.VMEM((DSPLIT, TILE), jnp.float32),
            pltpu.VMEM((DSPLIT, TILE), jnp.float32),
            pltpu.VMEM((DSPLIT, TILE), jnp.float32),
            pltpu.VMEM_SHARED((N, TILE), jnp.float32),
            pltpu.SemaphoreType.DMA,
            pltpu.SemaphoreType.DMA,
        ],
    )(_sc_body)
    halves = k(row, row_refs, row_embeddings.T)
    return jnp.concatenate([halves[0], halves[1]], axis=1)


# speedup vs baseline: 1.0036x; 1.0036x over previous
import functools

import jax
import jax.numpy as jnp
from jax import lax
from jax.experimental import pallas as pl
from jax.experimental.pallas import tpu as pltpu
from jax.experimental.pallas import tpu_sc as plsc

N = 512
D = 64
NCORE = 2
DSPLIT = D // NCORE
NSUB = 16
CHUNK = N // NSUB
LANES = 16
TILE = 128
BATCH = 4


def _sc_body(row_hbm, refs_hbm, embt_hbm, out_hbm,
             refs_v, rowbuf_v, ids_v, rows_v, tmp_v, out32_v,
             blk0, blk1, blk2, blk3, blk4, blk5, blk6, blk7,
             acc_sh, sem, sem_b):
    blksets = [[blk0, blk1, blk2, blk3], [blk4, blk5, blk6, blk7]]
    sems = [sem, sem_b]
    cid = lax.axis_index("c")
    sid = lax.axis_index("s")
    base = sid * CHUNK
    dlo = pl.multiple_of(cid * DSPLIT, DSPLIT)

    pltpu.sync_copy(refs_hbm.at[pl.ds(base, CHUNK)], refs_v)

    pltpu.async_copy(row_hbm.at[refs_v], rowbuf_v, sem).wait()
    for c in range(CHUNK // LANES):
        lane = lax.iota(jnp.int32, LANES)
        row_idx = lane + jnp.int32(c * LANES)
        col_idx = row_idx + base
        ids_v[pl.ds(LANES + c * LANES, LANES)] = plsc.load_gather(
            rowbuf_v, [row_idx, col_idx])

    ngroups = CHUNK // BATCH

    def fire(g):
        voffs, copies = [], []
        for b in range(BATCH):
            r = g * BATCH + b
            vid = ids_v[pl.ds(LANES + r, LANES)][0]
            vblk = vid // TILE
            voffs.append(vid - vblk * TILE)
            start = pl.multiple_of(vblk * TILE, TILE)
            copies.append(pltpu.async_copy(
                embt_hbm.at[pl.ds(dlo, DSPLIT), pl.ds(start, TILE)],
                blksets[g % 2][b], sems[g % 2]))
        return voffs, copies

    inflight = fire(0)
    for g in range(ngroups):
        voffs, copies = inflight
        if g + 1 < ngroups:
            nxt = fire(g + 1)
        for cp in copies:
            cp.wait()
        for b in range(BATCH):
            r = g * BATCH + b
            voff_vec = jnp.full((LANES,), voffs[b], jnp.int32)
            for c in range(DSPLIT // LANES):
                dvec = lax.iota(jnp.int32, LANES) + jnp.int32(c * LANES)
                rows_v[r, pl.ds(c * LANES, LANES)] = plsc.load_gather(
                    blksets[g % 2][b], [dvec, voff_vec])
            for c in range(DSPLIT // LANES, TILE // LANES):
                rows_v[r, pl.ds(c * LANES, LANES)] = jnp.zeros(
                    (LANES,), jnp.float32)
        if g + 1 < ngroups:
            inflight = nxt

    for r in range(CHUNK):
        for c in range(TILE // LANES):
            tmp_v[r, pl.ds(c * LANES, LANES)] = jnp.zeros((LANES,), jnp.float32)
    pltpu.sync_copy(tmp_v, acc_sh.at[pl.ds(base, CHUNK)])
    plsc.subcore_barrier()

    pltpu.sync_copy(rows_v, acc_sh.at[refs_v], add=True)
    plsc.subcore_barrier()

    pltpu.sync_copy(acc_sh.at[pl.ds(base, CHUNK)], tmp_v)
    for r in range(CHUNK):
        for c in range(DSPLIT // LANES):
            out32_v[r, pl.ds(c * LANES, LANES)] = \
                tmp_v[r, pl.ds(c * LANES, LANES)]
    pltpu.sync_copy(out32_v, out_hbm.at[cid, pl.ds(base, CHUNK)])


def kernel(row, row_refs, row_embeddings):
    mesh = plsc.VectorSubcoreMesh(core_axis_name="c", subcore_axis_name="s")
    k = functools.partial(
        pl.kernel,
        out_type=jax.ShapeDtypeStruct((NCORE, N, DSPLIT), jnp.float32),
        mesh=mesh,
        compiler_params=pltpu.CompilerParams(
            use_tc_tiling_on_sc=True, needs_layout_passes=False,
            disable_bounds_checks=True),
        scratch_types=[
            pltpu.VMEM((CHUNK,), jnp.int32),
            pltpu.VMEM((CHUNK, N), jnp.int32),
            pltpu.VMEM((2 * LANES + CHUNK,), jnp.int32),
            pltpu.VMEM((CHUNK, TILE), jnp.float32),
            pltpu.VMEM((CHUNK, TILE), jnp.float32),
            pltpu.VMEM((CHUNK, DSPLIT), jnp.float32),
            pltpu.VMEM((DSPLIT, TILE), jnp.float32),
            pltpu.VMEM((DSPLIT, TILE), jnp.float32),
            pltpu.VMEM((DSPLIT, TILE), jnp.float32),
            pltpu.VMEM((DSPLIT, TILE), jnp.float32),
            pltpu.VMEM((DSPLIT, TILE), jnp.float32),
            pltpu.VMEM((DSPLIT, TILE), jnp.float32),
            pltpu.VMEM((DSPLIT, TILE), jnp.float32),
            pltpu.VMEM((DSPLIT, TILE), jnp.float32),
            pltpu.VMEM_SHARED((N, TILE), jnp.float32),
            pltpu.SemaphoreType.DMA,
            pltpu.SemaphoreType.DMA,
        ],
    )(_sc_body)
    halves = k(row, row_refs, row_embeddings.T)
    return jnp.concatenate([halves[0], halves[1]], axis=1)
